# baseline (speedup 1.0000x reference)
import functools
import jax
import jax.numpy as jnp
from jax import lax
from jax.experimental import pallas as pl
from jax.experimental.pallas import tpu as pltpu
from jax.experimental.pallas import tpu_sc as plsc

_ROW_BLOCK = 1024
_ALPHA = 5.0
_BIG = 3.0e38
_L = 16


def _dist_kernel(rows_ref, sq_ref, all_ref, out_ref):
    rows = rows_ref[0]
    brow = rows.astype(jnp.bfloat16)
    brow2 = brow + brow
    ball = all_ref[0].astype(jnp.bfloat16)
    g2 = jax.lax.dot_general(
        brow2, ball, (((1,), (1,)), ((), ())),
        preferred_element_type=jnp.float32)
    sqr = (rows[:, 0:1] * rows[:, 0:1] + rows[:, 1:2] * rows[:, 1:2]
           + rows[:, 2:3] * rows[:, 2:3])
    sqc = sq_ref[0]
    w = g2.shape[1] // 2
    d2a = (sqr + sqc[:, :w]) - g2[:, :w]
    d2b = (sqr + sqc[:, w:]) - g2[:, w:]
    m1v = jnp.minimum(d2a, d2b)
    m2v = jnp.maximum(d2a, d2b)
    w //= 2
    while w >= 128:
        a1, b1 = m1v[:, :w], m1v[:, w:]
        a2, b2 = m2v[:, :w], m2v[:, w:]
        nhi = jnp.maximum(a1, b1)
        m1v = jnp.minimum(a1, b1)
        m2v = jnp.minimum(jnp.minimum(a2, b2), nhi)
        w //= 2
    m1 = jnp.min(m1v, axis=1, keepdims=True)
    eqv = m1v == m1
    cntv = jnp.sum(jnp.where(eqv, 1.0, 0.0), axis=1, keepdims=True)
    gtv = jnp.min(jnp.where(eqv, _BIG, m1v), axis=1, keepdims=True)
    partner = jnp.min(jnp.where(eqv, m2v, _BIG), axis=1, keepdims=True)
    sec = jnp.where(cntv >= 2.0, m1, gtv)
    m2 = jnp.minimum(sec, partner)
    dist = jnp.sqrt(jnp.maximum(m1, 1e-12)) + jnp.sqrt(jnp.maximum(m2, 1e-12))
    out_ref[0] = dist


def _sc_loss(dist_hbm, out_hbm, dist_v, rot_v, out_v, *, B, N):
    wid = lax.axis_index("s") * 2 + lax.axis_index("c")

    def allreduce(a):
        for k in (1, 2, 4, 8):
            rot_v[pl.ds(0, _L)] = a
            rot_v[pl.ds(_L, _L)] = a
            a = a + rot_v[pl.ds(k, _L)]
        return a

    @pl.when(wid == 0)
    def _():
        pltpu.sync_copy(dist_hbm, dist_v)
        nchunk = N // _L
        total = jnp.zeros((_L,), jnp.float32)
        for b in range(B):
            base = b * N

            def sum_body(i, acc):
                a = acc
                for j in range(4):
                    a = a + dist_v[pl.ds(base + (i * 4 + j) * _L, _L)]
                return a
            acc = lax.fori_loop(0, nchunk // 4, sum_body,
                                jnp.zeros((_L,), jnp.float32))
            thr = allreduce(acc) * (_ALPHA / N)

            def mask_body(i, acc2):
                a = acc2
                for j in range(4):
                    v = dist_v[pl.ds(base + (i * 4 + j) * _L, _L)]
                    a = a + jnp.where(v > thr, v, 0.0)
                return a
            total = total + lax.fori_loop(0, nchunk // 4, mask_body,
                                          jnp.zeros((_L,), jnp.float32))
        out_v[...] = allreduce(total)
        pltpu.sync_copy(out_v, out_hbm)


def kernel(xyz):
    B, N, _ = xyz.shape
    R = _ROW_BLOCK
    sq = jnp.sum(xyz * xyz, axis=-1)[:, None, :]
    dist = pl.pallas_call(
        _dist_kernel,
        grid=(B, N // R),
        in_specs=[
            pl.BlockSpec((1, R, 3), lambda b, i: (b, i, 0)),
            pl.BlockSpec((1, 1, N), lambda b, i: (b, 0, 0)),
            pl.BlockSpec((1, N, 3), lambda b, i: (b, 0, 0)),
        ],
        out_specs=pl.BlockSpec((1, R, 1), lambda b, i: (b, i, 0)),
        out_shape=jax.ShapeDtypeStruct((B, N, 1), jnp.float32),
    )(xyz, sq, xyz)

    mesh = plsc.VectorSubcoreMesh(core_axis_name="c", subcore_axis_name="s")
    loss_fn = functools.partial(
        pl.kernel,
        mesh=mesh,
        out_type=jax.ShapeDtypeStruct((_L,), jnp.float32),
        scratch_types=[
            pltpu.VMEM((B * N,), jnp.float32),
            pltpu.VMEM((2 * _L,), jnp.float32),
            pltpu.VMEM((_L,), jnp.float32),
        ],
    )(functools.partial(_sc_loss, B=B, N=N))
    loss = loss_fn(dist.reshape(B * N))
    return loss[0]


# speedup vs baseline: 1.3936x; 1.3936x over previous
import functools
import jax
import jax.numpy as jnp
from jax.experimental import pallas as pl
from jax.experimental.pallas import tpu as pltpu

_ROW_BLOCK = 1024
_ALPHA = 5.0
_BIG = 3.0e38


def _fused_kernel(rows_ref, all_ref, out_ref, dist_ref, sqc_ref, *, nb):
    b = pl.program_id(0)
    i = pl.program_id(1)
    rows = rows_ref[0]

    @pl.when(i == 0)
    def _sqcol():
        a = all_ref[0]
        s = (a[:, 0:1] * a[:, 0:1] + a[:, 1:2] * a[:, 1:2]
             + a[:, 2:3] * a[:, 2:3])
        sqc_ref[...] = s.T
    brow = rows.astype(jnp.bfloat16)
    brow2 = brow + brow
    ball = all_ref[0].astype(jnp.bfloat16)
    g2 = jax.lax.dot_general(
        brow2, ball,
        (((1,), (1,)), ((), ())),
        preferred_element_type=jnp.float32,
    )
    sqr = (rows[:, 0:1] * rows[:, 0:1] + rows[:, 1:2] * rows[:, 1:2]
           + rows[:, 2:3] * rows[:, 2:3])
    sqc = sqc_ref[...]
    w = g2.shape[1] // 2
    d2a = (sqr + sqc[:, :w]) - g2[:, :w]
    d2b = (sqr + sqc[:, w:]) - g2[:, w:]
    m1v = jnp.minimum(d2a, d2b)
    m2v = jnp.maximum(d2a, d2b)
    w //= 2
    while w >= 128:
        a1, b1 = m1v[:, :w], m1v[:, w:]
        a2, b2 = m2v[:, :w], m2v[:, w:]
        nhi = jnp.maximum(a1, b1)
        m1v = jnp.minimum(a1, b1)
        m2v = jnp.minimum(jnp.minimum(a2, b2), nhi)
        w //= 2
    m1 = jnp.min(m1v, axis=1, keepdims=True)
    eqv = m1v == m1
    cntv = jnp.sum(jnp.where(eqv, 1.0, 0.0), axis=1, keepdims=True)
    gtv = jnp.min(jnp.where(eqv, _BIG, m1v), axis=1, keepdims=True)
    partner = jnp.min(jnp.where(eqv, m2v, _BIG), axis=1, keepdims=True)
    sec = jnp.where(cntv >= 2.0, m1, gtv)
    m2 = jnp.minimum(sec, partner)
    dist = jnp.sqrt(jnp.maximum(m1, 1e-12)) + jnp.sqrt(jnp.maximum(m2, 1e-12))
    dist_ref[b, pl.ds(i * dist.shape[0], dist.shape[0])] = dist

    @pl.when(jnp.logical_and(b == dist_ref.shape[0] - 1, i == nb - 1))
    def _finalize():
        d = dist_ref[...]
        n = d.shape[1]
        avg = jnp.sum(d, axis=1, keepdims=True) / n
        masked = jnp.where(d > avg * _ALPHA, d, 0.0)
        out_ref[...] = jnp.sum(masked).reshape(1, 1)


def kernel(xyz):
    B, N, _ = xyz.shape
    R = _ROW_BLOCK
    nb = N // R
    loss = pl.pallas_call(
        functools.partial(_fused_kernel, nb=nb),
        grid=(B, nb),
        in_specs=[
            pl.BlockSpec((1, R, 3), lambda b, i: (b, i, 0)),
            pl.BlockSpec((1, N, 3), lambda b, i: (b, 0, 0)),
        ],
        out_specs=pl.BlockSpec((1, 1), lambda b, i: (0, 0)),
        out_shape=jax.ShapeDtypeStruct((1, 1), jnp.float32),
        scratch_shapes=[pltpu.VMEM((B, N, 1), jnp.float32),
                        pltpu.VMEM((1, N), jnp.float32)],
    )(xyz, xyz)
    return loss[0, 0]


# speedup vs baseline: 1.4843x; 1.0651x over previous
import functools
import jax
import jax.numpy as jnp
from jax.experimental import pallas as pl
from jax.experimental.pallas import tpu as pltpu

_ROW_BLOCK = 2048
_ALPHA = 5.0
_BIG = 3.0e38


def _fused_kernel(rows_ref, sq_ref, all_ref, out_ref, dist_ref, *, nb):
    b = pl.program_id(0)
    i = pl.program_id(1)
    rows = rows_ref[0]
    brow = rows.astype(jnp.bfloat16)
    brow2 = brow + brow
    ball = all_ref[0].astype(jnp.bfloat16)
    g2 = jax.lax.dot_general(
        brow2, ball,
        (((1,), (1,)), ((), ())),
        preferred_element_type=jnp.float32,
    )
    sqr = (rows[:, 0:1] * rows[:, 0:1] + rows[:, 1:2] * rows[:, 1:2]
           + rows[:, 2:3] * rows[:, 2:3])
    sqc = sq_ref[0]
    w = g2.shape[1] // 2
    d2a = (sqr + sqc[:, :w]) - g2[:, :w]
    d2b = (sqr + sqc[:, w:]) - g2[:, w:]
    m1v = jnp.minimum(d2a, d2b)
    m2v = jnp.maximum(d2a, d2b)
    w //= 2
    while w >= 128:
        a1, b1 = m1v[:, :w], m1v[:, w:]
        a2, b2 = m2v[:, :w], m2v[:, w:]
        nhi = jnp.maximum(a1, b1)
        m1v = jnp.minimum(a1, b1)
        m2v = jnp.minimum(jnp.minimum(a2, b2), nhi)
        w //= 2
    m1 = jnp.min(m1v, axis=1, keepdims=True)
    eqv = m1v == m1
    cntv = jnp.sum(jnp.where(eqv, 1.0, 0.0), axis=1, keepdims=True)
    gtv = jnp.min(jnp.where(eqv, _BIG, m1v), axis=1, keepdims=True)
    partner = jnp.min(jnp.where(eqv, m2v, _BIG), axis=1, keepdims=True)
    sec = jnp.where(cntv >= 2.0, m1, gtv)
    m2 = jnp.minimum(sec, partner)
    dist = jnp.sqrt(jnp.maximum(m1, 1e-12)) + jnp.sqrt(jnp.maximum(m2, 1e-12))
    dist_ref[b, pl.ds(i * dist.shape[0], dist.shape[0])] = dist

    @pl.when(jnp.logical_and(b == dist_ref.shape[0] - 1, i == nb - 1))
    def _finalize():
        d = dist_ref[...]
        n = d.shape[1]
        avg = jnp.sum(d, axis=1, keepdims=True) / n
        masked = jnp.where(d > avg * _ALPHA, d, 0.0)
        out_ref[...] = jnp.sum(masked).reshape(1, 1)


def kernel(xyz):
    B, N, _ = xyz.shape
    R = _ROW_BLOCK
    nb = N // R
    sq = jnp.sum(xyz * xyz, axis=-1)[:, None, :]
    loss = pl.pallas_call(
        functools.partial(_fused_kernel, nb=nb),
        grid=(B, nb),
        in_specs=[
            pl.BlockSpec((1, R, 3), lambda b, i: (b, i, 0)),
            pl.BlockSpec((1, 1, N), lambda b, i: (b, 0, 0)),
            pl.BlockSpec((1, N, 3), lambda b, i: (b, 0, 0)),
        ],
        out_specs=pl.BlockSpec((1, 1), lambda b, i: (0, 0)),
        out_shape=jax.ShapeDtypeStruct((1, 1), jnp.float32),
        scratch_shapes=[pltpu.VMEM((B, N, 1), jnp.float32)],
    )(xyz, sq, xyz)
    return loss[0, 0]
